# baseline (speedup 1.0000x reference)
import functools

import jax
import jax.numpy as jnp
from jax import lax
from jax.experimental import pallas as pl
from jax.experimental.pallas import tpu as pltpu
from jax.experimental.pallas import tpu_sc as plsc

NUM_CORES = 2
NUM_SUBCORES = 16
NW = NUM_CORES * NUM_SUBCORES

KPAD = 128
KSTAGE = 256
CHUNK = 128
NBUF = 4
NB2 = 2 * NBUF


def _gather_kernel(rows_per_w, f, d, keys_hbm, table_hbm, out_hbm, kv_pad,
                   kidx, rows_v, gsem, wsem):
    n_keys = rows_per_w * f
    n_chunks = n_keys // CHUNK
    wid = lax.axis_index("s") * NUM_CORES + lax.axis_index("c")
    base = wid * rows_per_w
    lane = lax.iota(jnp.int32, 16)

    for half in range(rows_per_w // KSTAGE):
        pltpu.sync_copy(
            keys_hbm.at[pl.ds(base + half * KSTAGE, KSTAGE)], kv_pad)
        kbase = half * KSTAGE * f

        @pl.loop(0, KSTAGE * f // 16)
        def _(j):
            flat = j * 16 + lane
            row = lax.shift_right_logical(flat * 20165, 19)
            col = flat - row * f
            vals = plsc.load_gather(kv_pad, [row, col])
            kidx[pl.ds(pl.multiple_of(kbase + j * 16, 16), 16)] = vals

    def start_gather(c, b):
        pltpu.async_copy(
            table_hbm.at[kidx.at[pl.ds(c * CHUNK, CHUNK)]],
            rows_v.at[b], gsem.at[b])

    def wait_gather(c, b):
        pltpu.make_async_copy(
            table_hbm.at[kidx.at[pl.ds(c * CHUNK, CHUNK)]],
            rows_v.at[b], gsem.at[b]).wait()

    def start_write(c, b):
        pltpu.async_copy(
            rows_v.at[b],
            out_hbm.at[pl.ds(base * f + c * CHUNK, CHUNK)], wsem.at[b])

    def wait_write(c, b):
        pltpu.make_async_copy(
            rows_v.at[b],
            out_hbm.at[pl.ds(base * f + c * CHUNK, CHUNK)], wsem.at[b]).wait()

    for b in range(NBUF):
        start_gather(b, b)

    for c in range(NBUF):
        wait_gather(c, c)
        start_write(c, c)
        start_gather(c + NBUF, c + NBUF)

    @pl.loop(NBUF, n_chunks - NBUF, step=NB2)
    def _(i):
        for k in range(NB2):
            c = i + k
            b = (NBUF + k) % NB2
            bn = (b + NBUF) % NB2
            wait_gather(c, b)
            start_write(c, b)
            wait_write(c - NBUF, bn)
            start_gather(c + NBUF, bn)

    for k in range(NBUF):
        c = n_chunks - NBUF + k
        b = c % NB2
        wait_gather(c, b)
        start_write(c, b)

    for j in range(NB2):
        c = n_chunks - NB2 + j
        wait_write(c, c % NB2)


def kernel(keys, table):
    b, f = keys.shape
    v, d = table.shape
    rows_per_w = b // NW
    keys_pad = jnp.pad(keys, ((0, 0), (0, KPAD - f)))

    mesh = plsc.VectorSubcoreMesh(core_axis_name="c", subcore_axis_name="s")
    out = pl.kernel(
        functools.partial(_gather_kernel, rows_per_w, f, d),
        out_type=jax.ShapeDtypeStruct((b * f, d), table.dtype),
        mesh=mesh,
        scratch_types=[
            pltpu.VMEM((KSTAGE, KPAD), jnp.int32),
            pltpu.VMEM((rows_per_w * f,), jnp.int32),
            pltpu.VMEM((NB2, CHUNK, d), jnp.float32),
            pltpu.SemaphoreType.DMA((NB2,)),
            pltpu.SemaphoreType.DMA((NB2,)),
        ],
        compiler_params=pltpu.CompilerParams(use_tc_tiling_on_sc=False, needs_layout_passes=False),
    )(keys_pad, table)
    return out.reshape(b, f, d)


# speedup vs baseline: 1.2281x; 1.2281x over previous
import functools

import jax
import jax.numpy as jnp
from jax import lax
from jax.experimental import pallas as pl
from jax.experimental.pallas import tpu as pltpu
from jax.experimental.pallas import tpu_sc as plsc

NUM_CORES = 2
NUM_SUBCORES = 16
NW = NUM_CORES * NUM_SUBCORES

KPAD = 128
KSTAGE = 256
CHUNK = 104
NBUF = 4
NB2 = 2 * NBUF


def _gather_kernel(rows_per_w, f, d, keys_hbm, table_hbm, out_hbm, kv_pad,
                   kidx, rows_v, gsem, wsem):
    n_keys = rows_per_w * f
    n_chunks = n_keys // CHUNK
    wid = lax.axis_index("s") * NUM_CORES + lax.axis_index("c")
    base = wid * rows_per_w
    lane = lax.iota(jnp.int32, 16)

    for half in range(rows_per_w // KSTAGE):
        pltpu.sync_copy(
            keys_hbm.at[pl.ds(base + half * KSTAGE, KSTAGE)], kv_pad)
        kbase = half * KSTAGE * f

        @pl.loop(0, KSTAGE * f // 16)
        def _(j):
            flat = j * 16 + lane
            row = lax.shift_right_logical(flat * 20165, 19)
            col = flat - row * f
            vals = plsc.load_gather(kv_pad, [row, col])
            kidx[pl.ds(pl.multiple_of(kbase + j * 16, 16), 16)] = vals

    def start_gather(c, b):
        pltpu.async_copy(
            table_hbm.at[kidx.at[pl.ds(c * CHUNK, CHUNK)]],
            rows_v.at[b], gsem.at[b])

    def wait_gather(c, b):
        pltpu.make_async_copy(
            table_hbm.at[kidx.at[pl.ds(c * CHUNK, CHUNK)]],
            rows_v.at[b], gsem.at[b]).wait()

    rows_per_chunk = CHUNK // f

    def start_write(c, b):
        for r in range(rows_per_chunk):
            pltpu.async_copy(
                rows_v.at[b].at[pl.ds(r * f, f)],
                out_hbm.at[base + c * rows_per_chunk + r, pl.ds(0, f),
                           pl.ds(0, d)],
                wsem.at[b])

    def wait_write(c, b):
        for r in range(rows_per_chunk):
            pltpu.make_async_copy(
                rows_v.at[b].at[pl.ds(r * f, f)],
                out_hbm.at[base + c * rows_per_chunk + r, pl.ds(0, f),
                           pl.ds(0, d)],
                wsem.at[b]).wait()

    for b in range(NBUF):
        start_gather(b, b)

    for c in range(NBUF):
        wait_gather(c, c)
        start_write(c, c)
        start_gather(c + NBUF, c + NBUF)

    @pl.loop(NBUF, n_chunks - NBUF, step=NB2)
    def _(i):
        for k in range(NB2):
            c = i + k
            b = (NBUF + k) % NB2
            bn = (b + NBUF) % NB2
            wait_gather(c, b)
            start_write(c, b)
            wait_write(c - NBUF, bn)
            start_gather(c + NBUF, bn)

    for k in range(NBUF):
        c = n_chunks - NBUF + k
        b = c % NB2
        wait_gather(c, b)
        start_write(c, b)

    for j in range(NB2):
        c = n_chunks - NB2 + j
        wait_write(c, c % NB2)


def kernel(keys, table):
    b, f = keys.shape
    v, d = table.shape
    rows_per_w = b // NW
    keys_pad = jnp.pad(keys, ((0, 0), (0, KPAD - f)))

    mesh = plsc.VectorSubcoreMesh(core_axis_name="c", subcore_axis_name="s")
    out = pl.kernel(
        functools.partial(_gather_kernel, rows_per_w, f, d),
        out_type=jax.ShapeDtypeStruct((b, 32, 128), table.dtype),
        mesh=mesh,
        scratch_types=[
            pltpu.VMEM((KSTAGE, KPAD), jnp.int32),
            pltpu.VMEM((rows_per_w * f,), jnp.int32),
            pltpu.VMEM((NB2, CHUNK, d), jnp.float32),
            pltpu.SemaphoreType.DMA((NB2,)),
            pltpu.SemaphoreType.DMA((NB2,)),
        ],
        compiler_params=pltpu.CompilerParams(use_tc_tiling_on_sc=False, needs_layout_passes=False),
    )(keys_pad, table)
    return out[:, :f, :d]
